# baseline (speedup 1.0000x reference)
import functools

import jax
import jax.numpy as jnp
from jax import lax
from jax.experimental import pallas as pl
from jax.experimental.pallas import tpu as pltpu
from jax.experimental.pallas import tpu_sc as plsc

N = 10000
DEG = 32
DN = 128
DE = 16
E = N * DEG
DOUT = 128
NMOL = 256

NC, NS, L = 2, 16, 16
NW = NC * NS
PER_W = 320
A = 8
CHUNKS = PER_W // A
ROWS = A * DEG
NCOL = DN // L
LAST_W = NW - 1
LAST_N = N - LAST_W * PER_W
LAST_CHUNKS = LAST_N // A

BLK = 2000
NB = N // BLK


def _make_sc_body(width):
    ncol = width // L

    def body(tbl, nidx, out, idxt, idx1, rows, dest, zrow, sacc,
             sg, swr, semz, sems):
        sid = lax.axis_index("s")
        wid = sid * NC + lax.axis_index("c")
        base = wid * PER_W
        nch = jnp.where(wid == LAST_W, LAST_CHUNKS, CHUNKS)

        @pl.when(wid != LAST_W)
        def _():
            pltpu.sync_copy(nidx.at[pl.ds(base, PER_W)], idxt)

        @pl.when(wid == LAST_W)
        def _():
            pltpu.sync_copy(nidx.at[pl.ds(LAST_W * PER_W, LAST_N)],
                            idxt.at[pl.ds(0, LAST_N)])

        def reformat(i, carry):
            for h in range(DEG // L):
                idx1[pl.ds(i * DEG + h * L, L)] = idxt[i, pl.ds(h * L, L)]
            return carry

        lax.fori_loop(0, PER_W, reformat, 0)

        def fire(g, buf):
            for h in range(2):
                sl = pl.ds(g * ROWS + h * 128, 128)
                dl = pl.ds(h * 128, 128)
                pltpu.async_copy(tbl.at[idx1.at[sl]], rows.at[buf].at[dl],
                                 sg.at[buf])

        def drain_gathers(g, buf):
            for h in range(2):
                sl = pl.ds(g * ROWS + h * 128, 128)
                dl = pl.ds(h * 128, 128)
                pltpu.make_async_copy(tbl.at[idx1.at[sl]],
                                     rows.at[buf].at[dl], sg.at[buf]).wait()

        def srow(buf):
            return sid * (2 * A) + buf * A

        def drain_write(g, buf):
            pltpu.make_async_copy(sacc.at[pl.ds(srow(buf), A)],
                                  out.at[pl.ds(base + g * A, A)],
                                  swr.at[buf]).wait()

        def drain_scatter(buf):
            pltpu.make_async_copy(rows.at[buf], sacc.at[dest.at[buf]],
                                  sems.at[buf]).wait()

        def issue_write(g, buf):
            pltpu.async_copy(sacc.at[pl.ds(srow(buf), A)],
                             out.at[pl.ds(base + g * A, A)], swr.at[buf])

        for i in range(A):
            for c in range(ncol):
                zrow[i, pl.ds(c * L, L)] = jnp.zeros((L,), jnp.float32)
        for buf in range(2):
            for v in range(ROWS // L):
                dest[buf, pl.ds(v * L, L)] = jnp.full(
                    (L,), srow(buf) + v // (DEG // L), jnp.int32)

        fire(0, 0)

        def chunk_body(g, carry):
            buf = lax.rem(g, 2)
            nbuf = lax.rem(g + 1, 2)

            @pl.when(g >= 1)
            def _():
                drain_scatter(nbuf)
                issue_write(g - 1, nbuf)

            @pl.when(g + 1 < nch)
            def _():
                fire(g + 1, nbuf)

            @pl.when(g >= 2)
            def _():
                drain_write(g - 2, buf)

            zc = pltpu.async_copy(zrow, sacc.at[pl.ds(srow(buf), A)],
                                  semz.at[buf])
            drain_gathers(g, buf)
            zc.wait()
            pltpu.async_copy(rows.at[buf], sacc.at[dest.at[buf]],
                             sems.at[buf], add=True)
            return carry

        lax.fori_loop(0, nch, chunk_body, 0)
        lbuf = lax.rem(nch - 1, 2)
        drain_scatter(lbuf)
        issue_write(nch - 1, lbuf)
        drain_write(nch - 2, lax.rem(nch - 2, 2))
        drain_write(nch - 1, lbuf)

    return body


@functools.cache
def _get_sc_kernel(width):
    mesh = plsc.VectorSubcoreMesh(
        core_axis_name="c", subcore_axis_name="s",
        num_cores=NC, num_subcores=NS)
    return pl.kernel(
        _make_sc_body(width),
        out_type=jax.ShapeDtypeStruct((N, width), jnp.float32),
        mesh=mesh,
        scratch_types=[
            pltpu.VMEM((PER_W, DEG), jnp.int32),
            pltpu.VMEM((PER_W * DEG,), jnp.int32),
            pltpu.VMEM((2, ROWS, width), jnp.float32),
            pltpu.VMEM((2, ROWS), jnp.int32),
            pltpu.VMEM((A, width), jnp.float32),
            pltpu.VMEM_SHARED((NS * 2 * A, width), jnp.float32),
            pltpu.SemaphoreType.DMA((2,)),
            pltpu.SemaphoreType.DMA((2,)),
            pltpu.SemaphoreType.DMA((2,)),
            pltpu.SemaphoreType.DMA((2,)),
        ],
        compiler_params=pltpu.CompilerParams(use_tc_tiling_on_sc=False),
    )


def _dot_t(x, w):
    return lax.dot_general(x, w, (((1,), (1,)), ((), ())),
                           preferred_element_type=jnp.float32)


def _onehot_t(mol_row):
    seg = lax.broadcasted_iota(jnp.int32, (NMOL, BLK), 0)
    return jnp.where(mol_row == seg, 1.0, 0.0)


def _tc1_body(ar_ref, asum_ref, bsum_ref, mol_ref, wdeg_ref, wself_ref,
              bias_ref, wout0_ref, bout0_ref, act_ref, stats_ref, fp0_ref):
    b = pl.program_id(0)
    ar = ar_ref[...]
    wdeg = wdeg_ref[...]
    wa = wdeg[:, :DN]
    wb = wdeg[:, DN:]
    wc = wa + wself_ref[...]
    act = (_dot_t(asum_ref[...], wa) + _dot_t(bsum_ref[...], wb)
           + _dot_t(ar, wc) + bias_ref[...])
    act_ref[...] = act

    psum = jnp.sum(act, axis=0, keepdims=True)
    psq = jnp.sum(act * act, axis=0, keepdims=True)

    logits = _dot_t(ar, wout0_ref[...]) + bout0_ref[...]
    m = jnp.max(logits, axis=1, keepdims=True)
    ex = jnp.exp(logits - m)
    soft = ex / jnp.sum(ex, axis=1, keepdims=True)
    oht = _onehot_t(mol_ref[0])
    fp_part = lax.dot_general(oht, soft, (((1,), (0,)), ((), ())),
                              preferred_element_type=jnp.float32)

    @pl.when(b == 0)
    def _():
        stats_ref[...] = jnp.zeros((2, DN), jnp.float32)
        fp0_ref[...] = jnp.zeros((NMOL, DOUT), jnp.float32)

    stats_ref[0:1, :] += psum
    stats_ref[1:2, :] += psq
    fp0_ref[...] += fp_part


_tc1 = pl.pallas_call(
    _tc1_body,
    grid=(NB,),
    in_specs=[
        pl.BlockSpec((BLK, DN), lambda b: (b, 0)),
        pl.BlockSpec((BLK, DN), lambda b: (b, 0)),
        pl.BlockSpec((BLK, DE), lambda b: (b, 0)),
        pl.BlockSpec((1, 1, BLK), lambda b: (b, 0, 0)),
        pl.BlockSpec((DOUT, DN + DE), lambda b: (0, 0)),
        pl.BlockSpec((DOUT, DN), lambda b: (0, 0)),
        pl.BlockSpec((1, DOUT), lambda b: (0, 0)),
        pl.BlockSpec((DOUT, DN), lambda b: (0, 0)),
        pl.BlockSpec((1, DOUT), lambda b: (0, 0)),
    ],
    out_specs=[
        pl.BlockSpec((BLK, DN), lambda b: (b, 0)),
        pl.BlockSpec((2, DN), lambda b: (0, 0)),
        pl.BlockSpec((NMOL, DOUT), lambda b: (0, 0)),
    ],
    out_shape=[
        jax.ShapeDtypeStruct((N, DN), jnp.float32),
        jax.ShapeDtypeStruct((2, DN), jnp.float32),
        jax.ShapeDtypeStruct((NMOL, DOUT), jnp.float32),
    ],
)


def _tc2_body(act_ref, mol_ref, stats_ref, fp0_ref, wout1_ref, bout1_ref,
              out_ref):
    b = pl.program_id(0)
    mean = stats_ref[0:1, :] * (1.0 / N)
    var = stats_ref[1:2, :] * (1.0 / N) - mean * mean
    h = jnp.maximum((act_ref[...] - mean) * lax.rsqrt(var + 1e-5), 0.0)
    logits = _dot_t(h, wout1_ref[...]) + bout1_ref[...]
    m = jnp.max(logits, axis=1, keepdims=True)
    ex = jnp.exp(logits - m)
    soft = ex / jnp.sum(ex, axis=1, keepdims=True)
    oht = _onehot_t(mol_ref[0])
    fp_part = lax.dot_general(oht, soft, (((1,), (0,)), ((), ())),
                              preferred_element_type=jnp.float32)

    @pl.when(b == 0)
    def _():
        out_ref[...] = fp0_ref[...]

    out_ref[...] += fp_part


_tc2 = pl.pallas_call(
    _tc2_body,
    grid=(NB,),
    in_specs=[
        pl.BlockSpec((BLK, DN), lambda b: (b, 0)),
        pl.BlockSpec((1, 1, BLK), lambda b: (b, 0, 0)),
        pl.BlockSpec((2, DN), lambda b: (0, 0)),
        pl.BlockSpec((NMOL, DOUT), lambda b: (0, 0)),
        pl.BlockSpec((DOUT, DOUT), lambda b: (0, 0)),
        pl.BlockSpec((1, DOUT), lambda b: (0, 0)),
    ],
    out_specs=pl.BlockSpec((NMOL, DOUT), lambda b: (0, 0)),
    out_shape=jax.ShapeDtypeStruct((NMOL, DOUT), jnp.float32),
)


def kernel(atom_repr, bond_repr, atom_nbr_idx, bond_nbr_idx, mol_ids,
           W_deg, W_self, bias, W_out0, b_out0, W_out1, b_out1):
    asum = _get_sc_kernel(DN)(atom_repr, atom_nbr_idx)
    bsum = _get_sc_kernel(DE)(bond_repr, bond_nbr_idx)

    mol3 = mol_ids.astype(jnp.int32).reshape(NB, 1, BLK)
    act, stats, fp0 = _tc1(atom_repr, asum, bsum, mol3, W_deg, W_self, bias,
                           W_out0, b_out0.reshape(1, DOUT))
    return _tc2(act, mol3, stats, fp0, W_out1, b_out1.reshape(1, DOUT))


# speedup vs baseline: 1.3433x; 1.3433x over previous
import functools

import jax
import jax.numpy as jnp
from jax import lax
from jax.experimental import pallas as pl
from jax.experimental.pallas import tpu as pltpu
from jax.experimental.pallas import tpu_sc as plsc

N = 10000
DEG = 32
DN = 128
DE = 16
E = N * DEG
DOUT = 128
NMOL = 256

NC, NS, L = 2, 16, 16
NW = NC * NS
PER_W = 320
A = 8
CHUNKS = PER_W // A
ROWS = A * DEG
NCOL = DN // L
LAST_W = NW - 1
LAST_N = N - LAST_W * PER_W
LAST_CHUNKS = LAST_N // A

BLK = 2000
NB = N // BLK


def _make_sc_body(width):
    ncol = width // L

    def body(tbl, nidx, dep, out, idxt, idx1, rows, dest, zrow, sacc,
             sg, swr, semz, sems):
        del dep
        sid = lax.axis_index("s")
        wid = sid * NC + lax.axis_index("c")
        base = wid * PER_W
        nch = jnp.where(wid == LAST_W, LAST_CHUNKS, CHUNKS)

        @pl.when(wid != LAST_W)
        def _():
            pltpu.sync_copy(nidx.at[pl.ds(base, PER_W)], idxt)

        @pl.when(wid == LAST_W)
        def _():
            pltpu.sync_copy(nidx.at[pl.ds(LAST_W * PER_W, LAST_N)],
                            idxt.at[pl.ds(0, LAST_N)])

        def reformat(i, carry):
            for h in range(DEG // L):
                idx1[pl.ds(i * DEG + h * L, L)] = idxt[i, pl.ds(h * L, L)]
            return carry

        lax.fori_loop(0, PER_W, reformat, 0)

        def fire(g, buf):
            for h in range(2):
                sl = pl.ds(g * ROWS + h * 128, 128)
                dl = pl.ds(h * 128, 128)
                pltpu.async_copy(tbl.at[idx1.at[sl]], rows.at[buf].at[dl],
                                 sg.at[buf])

        def drain_gathers(g, buf):
            for h in range(2):
                sl = pl.ds(g * ROWS + h * 128, 128)
                dl = pl.ds(h * 128, 128)
                pltpu.make_async_copy(tbl.at[idx1.at[sl]],
                                     rows.at[buf].at[dl], sg.at[buf]).wait()

        def srow(buf):
            return sid * (2 * A) + buf * A

        def drain_write(g, buf):
            pltpu.make_async_copy(sacc.at[pl.ds(srow(buf), A)],
                                  out.at[pl.ds(base + g * A, A)],
                                  swr.at[buf]).wait()

        def drain_scatter(buf):
            pltpu.make_async_copy(rows.at[buf], sacc.at[dest.at[buf]],
                                  sems.at[buf]).wait()

        def issue_write(g, buf):
            pltpu.async_copy(sacc.at[pl.ds(srow(buf), A)],
                             out.at[pl.ds(base + g * A, A)], swr.at[buf])

        for i in range(A):
            for c in range(ncol):
                zrow[i, pl.ds(c * L, L)] = jnp.zeros((L,), jnp.float32)
        for buf in range(2):
            for v in range(ROWS // L):
                dest[buf, pl.ds(v * L, L)] = jnp.full(
                    (L,), srow(buf) + v // (DEG // L), jnp.int32)

        fire(0, 0)

        def chunk_body(g, carry):
            buf = lax.rem(g, 2)
            nbuf = lax.rem(g + 1, 2)

            @pl.when(g >= 1)
            def _():
                drain_scatter(nbuf)
                issue_write(g - 1, nbuf)

            @pl.when(g + 1 < nch)
            def _():
                fire(g + 1, nbuf)

            @pl.when(g >= 2)
            def _():
                drain_write(g - 2, buf)

            zc = pltpu.async_copy(zrow, sacc.at[pl.ds(srow(buf), A)],
                                  semz.at[buf])
            drain_gathers(g, buf)
            zc.wait()
            pltpu.async_copy(rows.at[buf], sacc.at[dest.at[buf]],
                             sems.at[buf], add=True)
            return carry

        lax.fori_loop(0, nch, chunk_body, 0)
        lbuf = lax.rem(nch - 1, 2)
        drain_scatter(lbuf)
        issue_write(nch - 1, lbuf)
        drain_write(nch - 2, lax.rem(nch - 2, 2))
        drain_write(nch - 1, lbuf)

    return body


@functools.cache
def _get_sc_kernel(width):
    mesh = plsc.VectorSubcoreMesh(
        core_axis_name="c", subcore_axis_name="s",
        num_cores=NC, num_subcores=NS)
    return pl.kernel(
        _make_sc_body(width),
        out_type=jax.ShapeDtypeStruct((N, width), jnp.float32),
        mesh=mesh,
        scratch_types=[
            pltpu.VMEM((PER_W, DEG), jnp.int32),
            pltpu.VMEM((PER_W * DEG,), jnp.int32),
            pltpu.VMEM((2, ROWS, width), jnp.float32),
            pltpu.VMEM((2, ROWS), jnp.int32),
            pltpu.VMEM((A, width), jnp.float32),
            pltpu.VMEM_SHARED((NS * 2 * A, width), jnp.float32),
            pltpu.SemaphoreType.DMA((2,)),
            pltpu.SemaphoreType.DMA((2,)),
            pltpu.SemaphoreType.DMA((2,)),
            pltpu.SemaphoreType.DMA((2,)),
        ],
        compiler_params=pltpu.CompilerParams(use_tc_tiling_on_sc=False),
    )


def _dot_t(x, w):
    return lax.dot_general(x, w, (((1,), (1,)), ((), ())),
                           preferred_element_type=jnp.float32)


def _onehot_t(mol_row):
    seg = lax.broadcasted_iota(jnp.int32, (NMOL, BLK), 0)
    return jnp.where(mol_row == seg, 1.0, 0.0)


def _tc1_body(ar_ref, asum_ref, bsum_ref, mol_ref, wdeg_ref, wself_ref,
              bias_ref, wout0_ref, bout0_ref, act_ref, stats_ref, fp0_ref):
    b = pl.program_id(0)
    ar = ar_ref[...]
    wdeg = wdeg_ref[...]
    wa = wdeg[:, :DN]
    wb = wdeg[:, DN:]
    wc = wa + wself_ref[...]
    act = (_dot_t(asum_ref[...], wa) + _dot_t(bsum_ref[...], wb)
           + _dot_t(ar, wc) + bias_ref[...])
    act_ref[...] = act

    psum = jnp.sum(act, axis=0, keepdims=True)
    psq = jnp.sum(act * act, axis=0, keepdims=True)

    logits = _dot_t(ar, wout0_ref[...]) + bout0_ref[...]
    m = jnp.max(logits, axis=1, keepdims=True)
    ex = jnp.exp(logits - m)
    soft = ex / jnp.sum(ex, axis=1, keepdims=True)
    oht = _onehot_t(mol_ref[0])
    fp_part = lax.dot_general(oht, soft, (((1,), (0,)), ((), ())),
                              preferred_element_type=jnp.float32)

    @pl.when(b == 0)
    def _():
        stats_ref[...] = jnp.zeros((2, DN), jnp.float32)
        fp0_ref[...] = jnp.zeros((NMOL, DOUT), jnp.float32)

    stats_ref[0:1, :] += psum
    stats_ref[1:2, :] += psq
    fp0_ref[...] += fp_part


_tc1 = pl.pallas_call(
    _tc1_body,
    grid=(NB,),
    in_specs=[
        pl.BlockSpec((BLK, DN), lambda b: (b, 0)),
        pl.BlockSpec((BLK, DN), lambda b: (b, 0)),
        pl.BlockSpec((BLK, DE), lambda b: (b, 0)),
        pl.BlockSpec((1, 1, BLK), lambda b: (b, 0, 0)),
        pl.BlockSpec((DOUT, DN + DE), lambda b: (0, 0)),
        pl.BlockSpec((DOUT, DN), lambda b: (0, 0)),
        pl.BlockSpec((1, DOUT), lambda b: (0, 0)),
        pl.BlockSpec((DOUT, DN), lambda b: (0, 0)),
        pl.BlockSpec((1, DOUT), lambda b: (0, 0)),
    ],
    out_specs=[
        pl.BlockSpec((BLK, DN), lambda b: (b, 0)),
        pl.BlockSpec((2, DN), lambda b: (0, 0)),
        pl.BlockSpec((NMOL, DOUT), lambda b: (0, 0)),
    ],
    out_shape=[
        jax.ShapeDtypeStruct((N, DN), jnp.float32),
        jax.ShapeDtypeStruct((2, DN), jnp.float32),
        jax.ShapeDtypeStruct((NMOL, DOUT), jnp.float32),
    ],
)


def _tc2_body(act_ref, mol_ref, stats_ref, fp0_ref, wout1_ref, bout1_ref,
              out_ref):
    b = pl.program_id(0)
    mean = stats_ref[0:1, :] * (1.0 / N)
    var = stats_ref[1:2, :] * (1.0 / N) - mean * mean
    h = jnp.maximum((act_ref[...] - mean) * lax.rsqrt(var + 1e-5), 0.0)
    logits = _dot_t(h, wout1_ref[...]) + bout1_ref[...]
    m = jnp.max(logits, axis=1, keepdims=True)
    ex = jnp.exp(logits - m)
    soft = ex / jnp.sum(ex, axis=1, keepdims=True)
    oht = _onehot_t(mol_ref[0])
    fp_part = lax.dot_general(oht, soft, (((1,), (0,)), ((), ())),
                              preferred_element_type=jnp.float32)

    @pl.when(b == 0)
    def _():
        out_ref[...] = fp0_ref[...]

    out_ref[...] += fp_part


_tc2 = pl.pallas_call(
    _tc2_body,
    grid=(NB,),
    in_specs=[
        pl.BlockSpec((BLK, DN), lambda b: (b, 0)),
        pl.BlockSpec((1, 1, BLK), lambda b: (b, 0, 0)),
        pl.BlockSpec((2, DN), lambda b: (0, 0)),
        pl.BlockSpec((NMOL, DOUT), lambda b: (0, 0)),
        pl.BlockSpec((DOUT, DOUT), lambda b: (0, 0)),
        pl.BlockSpec((1, DOUT), lambda b: (0, 0)),
    ],
    out_specs=pl.BlockSpec((NMOL, DOUT), lambda b: (0, 0)),
    out_shape=jax.ShapeDtypeStruct((NMOL, DOUT), jnp.float32),
)


def kernel(atom_repr, bond_repr, atom_nbr_idx, bond_nbr_idx, mol_ids,
           W_deg, W_self, bias, W_out0, b_out0, W_out1, b_out1):
    asum = _get_sc_kernel(DN)(atom_repr, atom_nbr_idx,
                              jnp.zeros((1, 1), jnp.float32))
    bsum = _get_sc_kernel(DE)(bond_repr, bond_nbr_idx,
                              lax.slice(asum, (0, 0), (1, 1)))

    mol3 = mol_ids.astype(jnp.int32).reshape(NB, 1, BLK)
    act, stats, fp0 = _tc1(atom_repr, asum, bsum, mol3, W_deg, W_self, bias,
                           W_out0, b_out0.reshape(1, DOUT))
    return _tc2(act, mol3, stats, fp0, W_out1, b_out1.reshape(1, DOUT))


# speedup vs baseline: 1.3546x; 1.0084x over previous
import functools

import jax
import jax.numpy as jnp
from jax import lax
from jax.experimental import pallas as pl
from jax.experimental.pallas import tpu as pltpu
from jax.experimental.pallas import tpu_sc as plsc

N = 10000
DEG = 32
DN = 128
DE = 16
E = N * DEG
DOUT = 128
NMOL = 256

NC, NS, L = 2, 16, 16
NW = NC * NS
PER_W = 320
A = 8
CHUNKS = PER_W // A
ROWS = A * DEG
NCOL = DN // L
LAST_W = NW - 1
LAST_N = N - LAST_W * PER_W
LAST_CHUNKS = LAST_N // A

BLK = 2000
NB = N // BLK


def _make_sc_body(width):
    ncol = width // L

    def body(tbl, nidx, dep, out, idxt, idx1, rows, dest, zrow, sacc,
             sg, swr, semz, sems):
        del dep
        sid = lax.axis_index("s")
        wid = sid * NC + lax.axis_index("c")
        base = wid * PER_W
        nch = jnp.where(wid == LAST_W, LAST_CHUNKS, CHUNKS)

        @pl.when(wid != LAST_W)
        def _():
            pltpu.sync_copy(nidx.at[pl.ds(base, PER_W)], idxt)

        @pl.when(wid == LAST_W)
        def _():
            pltpu.sync_copy(nidx.at[pl.ds(LAST_W * PER_W, LAST_N)],
                            idxt.at[pl.ds(0, LAST_N)])

        def reformat(i, carry):
            for h in range(DEG // L):
                idx1[pl.ds(i * DEG + h * L, L)] = idxt[i, pl.ds(h * L, L)]
            return carry

        lax.fori_loop(0, PER_W, reformat, 0)

        def fire(g, buf):
            for h in range(2):
                sl = pl.ds(g * ROWS + h * 128, 128)
                dl = pl.ds(h * 128, 128)
                pltpu.async_copy(tbl.at[idx1.at[sl]], rows.at[buf].at[dl],
                                 sg.at[buf])

        def drain_gathers(g, buf):
            for h in range(2):
                sl = pl.ds(g * ROWS + h * 128, 128)
                dl = pl.ds(h * 128, 128)
                pltpu.make_async_copy(tbl.at[idx1.at[sl]],
                                     rows.at[buf].at[dl], sg.at[buf]).wait()

        def srow(buf):
            return sid * (2 * A) + buf * A

        def out_dst(g):
            if width == DN:
                return out.at[pl.ds(base + g * A, A)]
            return out.at[pl.ds(base + g * A, A), pl.ds(0, width)]

        def drain_write(g, buf):
            pltpu.make_async_copy(sacc.at[pl.ds(srow(buf), A)],
                                  out_dst(g), swr.at[buf]).wait()

        def drain_scatter(buf):
            pltpu.make_async_copy(rows.at[buf], sacc.at[dest.at[buf]],
                                  sems.at[buf]).wait()

        def issue_write(g, buf):
            pltpu.async_copy(sacc.at[pl.ds(srow(buf), A)],
                             out_dst(g), swr.at[buf])

        for i in range(A):
            for c in range(ncol):
                zrow[i, pl.ds(c * L, L)] = jnp.zeros((L,), jnp.float32)
        for buf in range(2):
            for v in range(ROWS // L):
                dest[buf, pl.ds(v * L, L)] = jnp.full(
                    (L,), srow(buf) + v // (DEG // L), jnp.int32)

        fire(0, 0)

        def chunk_body(g, carry):
            buf = lax.rem(g, 2)
            nbuf = lax.rem(g + 1, 2)

            @pl.when(g >= 1)
            def _():
                drain_scatter(nbuf)
                issue_write(g - 1, nbuf)

            @pl.when(g + 1 < nch)
            def _():
                fire(g + 1, nbuf)

            @pl.when(g >= 2)
            def _():
                drain_write(g - 2, buf)

            zc = pltpu.async_copy(zrow, sacc.at[pl.ds(srow(buf), A)],
                                  semz.at[buf])
            drain_gathers(g, buf)
            zc.wait()
            pltpu.async_copy(rows.at[buf], sacc.at[dest.at[buf]],
                             sems.at[buf], add=True)
            return carry

        lax.fori_loop(0, nch, chunk_body, 0)
        lbuf = lax.rem(nch - 1, 2)
        drain_scatter(lbuf)
        issue_write(nch - 1, lbuf)
        drain_write(nch - 2, lax.rem(nch - 2, 2))
        drain_write(nch - 1, lbuf)

    return body


@functools.cache
def _get_sc_kernel(width):
    mesh = plsc.VectorSubcoreMesh(
        core_axis_name="c", subcore_axis_name="s",
        num_cores=NC, num_subcores=NS)
    return pl.kernel(
        _make_sc_body(width),
        out_type=jax.ShapeDtypeStruct((N, DN), jnp.float32),
        mesh=mesh,
        scratch_types=[
            pltpu.VMEM((PER_W, DN), jnp.int32),
            pltpu.VMEM((PER_W * DEG,), jnp.int32),
            pltpu.VMEM((2, ROWS, width), jnp.float32),
            pltpu.VMEM((2, ROWS), jnp.int32),
            pltpu.VMEM((A, width), jnp.float32),
            pltpu.VMEM_SHARED((NS * 2 * A, width), jnp.float32),
            pltpu.SemaphoreType.DMA((2,)),
            pltpu.SemaphoreType.DMA((2,)),
            pltpu.SemaphoreType.DMA((2,)),
            pltpu.SemaphoreType.DMA((2,)),
        ],
        compiler_params=pltpu.CompilerParams(use_tc_tiling_on_sc=False),
    )


def _dot_t(x, w):
    return lax.dot_general(x, w, (((1,), (1,)), ((), ())),
                           preferred_element_type=jnp.float32)


def _onehot_t(mol_row):
    seg = lax.broadcasted_iota(jnp.int32, (NMOL, BLK), 0)
    return jnp.where(mol_row == seg, 1.0, 0.0)


def _tc1_body(ar_ref, asum_ref, bsum_ref, mol_ref, wdeg_ref, wself_ref,
              bias_ref, wout0_ref, bout0_ref, act_ref, stats_ref, fp0_ref):
    b = pl.program_id(0)
    ar = ar_ref[...]
    wdeg = wdeg_ref[...]
    wa = wdeg[:, :DN]
    wb = wdeg[:, DN:]
    wc = wa + wself_ref[...]
    act = (_dot_t(asum_ref[...], wa) + _dot_t(bsum_ref[...][:, :DE], wb)
           + _dot_t(ar, wc) + bias_ref[...])
    act_ref[...] = act

    psum = jnp.sum(act, axis=0, keepdims=True)
    psq = jnp.sum(act * act, axis=0, keepdims=True)

    logits = _dot_t(ar, wout0_ref[...]) + bout0_ref[...]
    m = jnp.max(logits, axis=1, keepdims=True)
    ex = jnp.exp(logits - m)
    soft = ex / jnp.sum(ex, axis=1, keepdims=True)
    oht = _onehot_t(mol_ref[0])
    fp_part = lax.dot_general(oht, soft, (((1,), (0,)), ((), ())),
                              preferred_element_type=jnp.float32)

    @pl.when(b == 0)
    def _():
        stats_ref[...] = jnp.zeros((2, DN), jnp.float32)
        fp0_ref[...] = jnp.zeros((NMOL, DOUT), jnp.float32)

    stats_ref[0:1, :] += psum
    stats_ref[1:2, :] += psq
    fp0_ref[...] += fp_part


_tc1 = pl.pallas_call(
    _tc1_body,
    grid=(NB,),
    in_specs=[
        pl.BlockSpec((BLK, DN), lambda b: (b, 0)),
        pl.BlockSpec((BLK, DN), lambda b: (b, 0)),
        pl.BlockSpec((BLK, DN), lambda b: (b, 0)),
        pl.BlockSpec((1, 1, BLK), lambda b: (b, 0, 0)),
        pl.BlockSpec((DOUT, DN + DE), lambda b: (0, 0)),
        pl.BlockSpec((DOUT, DN), lambda b: (0, 0)),
        pl.BlockSpec((1, DOUT), lambda b: (0, 0)),
        pl.BlockSpec((DOUT, DN), lambda b: (0, 0)),
        pl.BlockSpec((1, DOUT), lambda b: (0, 0)),
    ],
    out_specs=[
        pl.BlockSpec((BLK, DN), lambda b: (b, 0)),
        pl.BlockSpec((2, DN), lambda b: (0, 0)),
        pl.BlockSpec((NMOL, DOUT), lambda b: (0, 0)),
    ],
    out_shape=[
        jax.ShapeDtypeStruct((N, DN), jnp.float32),
        jax.ShapeDtypeStruct((2, DN), jnp.float32),
        jax.ShapeDtypeStruct((NMOL, DOUT), jnp.float32),
    ],
)


def _tc2_body(act_ref, mol_ref, stats_ref, fp0_ref, wout1_ref, bout1_ref,
              out_ref):
    b = pl.program_id(0)
    mean = stats_ref[0:1, :] * (1.0 / N)
    var = stats_ref[1:2, :] * (1.0 / N) - mean * mean
    h = jnp.maximum((act_ref[...] - mean) * lax.rsqrt(var + 1e-5), 0.0)
    logits = _dot_t(h, wout1_ref[...]) + bout1_ref[...]
    m = jnp.max(logits, axis=1, keepdims=True)
    ex = jnp.exp(logits - m)
    soft = ex / jnp.sum(ex, axis=1, keepdims=True)
    oht = _onehot_t(mol_ref[0])
    fp_part = lax.dot_general(oht, soft, (((1,), (0,)), ((), ())),
                              preferred_element_type=jnp.float32)

    @pl.when(b == 0)
    def _():
        out_ref[...] = fp0_ref[...]

    out_ref[...] += fp_part


_tc2 = pl.pallas_call(
    _tc2_body,
    grid=(NB,),
    in_specs=[
        pl.BlockSpec((BLK, DN), lambda b: (b, 0)),
        pl.BlockSpec((1, 1, BLK), lambda b: (b, 0, 0)),
        pl.BlockSpec((2, DN), lambda b: (0, 0)),
        pl.BlockSpec((NMOL, DOUT), lambda b: (0, 0)),
        pl.BlockSpec((DOUT, DOUT), lambda b: (0, 0)),
        pl.BlockSpec((1, DOUT), lambda b: (0, 0)),
    ],
    out_specs=pl.BlockSpec((NMOL, DOUT), lambda b: (0, 0)),
    out_shape=jax.ShapeDtypeStruct((NMOL, DOUT), jnp.float32),
)


def kernel(atom_repr, bond_repr, atom_nbr_idx, bond_nbr_idx, mol_ids,
           W_deg, W_self, bias, W_out0, b_out0, W_out1, b_out1):
    aidx_p = jnp.pad(atom_nbr_idx, ((0, 0), (0, DN - DEG)))
    bidx_p = jnp.pad(bond_nbr_idx, ((0, 0), (0, DN - DEG)))
    asum = _get_sc_kernel(DN)(atom_repr, aidx_p,
                              jnp.zeros((8, DN), jnp.float32))
    bsum = _get_sc_kernel(DE)(bond_repr, bidx_p,
                              lax.slice(asum, (0, 0), (8, DN)))

    mol3 = mol_ids.astype(jnp.int32).reshape(NB, 1, BLK)
    act, stats, fp0 = _tc1(atom_repr, asum, bsum, mol3, W_deg, W_self, bias,
                           W_out0, b_out0.reshape(1, DOUT))
    return _tc2(act, mol3, stats, fp0, W_out1, b_out1.reshape(1, DOUT))
